# Initial kernel scaffold; baseline (speedup 1.0000x reference)
import jax
import jax.numpy as jnp
from jax.experimental import pallas as pl


def kernel(x, masked_x, pos_enc, Wc_self, Wc_nbr, bc, Wt_self, Wt_nbr, bt, Wp1, bp1, Wp2, bp2, edge_index, target_nodes):
    raise NotImplementedError("write your pallas kernel here")


# speedup vs baseline: 125.9811x; 125.9811x over previous
import functools

import jax
import jax.numpy as jnp
from jax.experimental import pallas as pl

_N = 10000
_NP = 10240
_D = 128
_PE = 16
_T = 64
_EMB = _D + _PE


def _mm_body(a_ref, b_ref, add_ref, o_ref, *, epilogue):
    k = pl.program_id(2)

    @pl.when(k == 0)
    def _():
        o_ref[...] = jnp.zeros_like(o_ref)

    o_ref[...] += jnp.dot(a_ref[...], b_ref[...],
                          preferred_element_type=jnp.float32)

    @pl.when(k == pl.num_programs(2) - 1)
    def _():
        acc = o_ref[...] + add_ref[...]
        if epilogue == 'relu':
            acc = jnp.maximum(acc, 0.0)
        elif epilogue == 'mask':
            acc = (acc > 0.0).astype(jnp.float32)
        o_ref[...] = acc


def _mm(a, b, add, epilogue, bm, bn, bk, add_is_mat):
    m, kdim = a.shape
    _, n = b.shape
    if add_is_mat:
        add_spec = pl.BlockSpec((bm, bn), lambda i, j, k: (i, j))
    else:
        add_spec = pl.BlockSpec((1, bn), lambda i, j, k: (0, j))
    return pl.pallas_call(
        functools.partial(_mm_body, epilogue=epilogue),
        grid=(m // bm, n // bn, kdim // bk),
        in_specs=[
            pl.BlockSpec((bm, bk), lambda i, j, k: (i, k)),
            pl.BlockSpec((bk, bn), lambda i, j, k: (k, j)),
            add_spec,
        ],
        out_specs=pl.BlockSpec((bm, bn), lambda i, j, k: (i, j)),
        out_shape=jax.ShapeDtypeStruct((m, n), jnp.float32),
    )(a, b, add)


def _ctx_body(g_ref, aself_ref, m_ref, w_ref, o_ref):
    v = pl.program_id(1)

    @pl.when(v == 0)
    def _():
        o_ref[...] = jnp.zeros_like(o_ref)

    rows = []
    for j in range(8):
        h = jnp.maximum(
            aself_ref[...] + jnp.dot(g_ref[:, j * _D:(j + 1) * _D],
                                     w_ref[...],
                                     preferred_element_type=jnp.float32),
            0.0)
        rows.append(jnp.dot(m_ref[j:j + 1, :], h,
                            preferred_element_type=jnp.float32))
    o_ref[...] += jnp.concatenate(rows, axis=0)


def _pred_body(c_ref, w1_ref, b1_ref, w2_ref, b2_ref, o_ref):
    hid = jnp.maximum(
        jnp.dot(c_ref[...], w1_ref[...],
                preferred_element_type=jnp.float32) + b1_ref[...], 0.0)
    o_ref[...] = jnp.dot(hid, w2_ref[...],
                         preferred_element_type=jnp.float32) + b2_ref[...]


def kernel(x, masked_x, pos_enc, Wc_self, Wc_nbr, bc, Wt_self, Wt_nbr, bt,
           Wp1, bp1, Wp2, bp2, edge_index, target_nodes):
    row, col = edge_index[0], edge_index[1]
    ones_e = jnp.ones(row.shape, jnp.float32)
    a_cnt = jnp.zeros((_NP, _NP), jnp.float32).at[row, col].add(ones_e)
    at_cnt = jnp.zeros((_NP, _NP), jnp.float32).at[col, row].add(ones_e)

    pad = _NP - _N
    xp = jnp.pad(x, ((0, pad), (0, 0)))
    mxp = jnp.pad(masked_x, ((0, pad), (0, 0)))

    z128 = jnp.zeros((1, 128), jnp.float32)

    agg_t = _mm(at_cnt, xp, z128, 'none', 1024, 128, 512, False)
    xcat = jnp.concatenate([xp, agg_t], axis=1)
    wt = jnp.concatenate([Wt_self, Wt_nbr], axis=0)
    target_x = _mm(xcat, wt, bt[None, :], 'relu', 1024, 128, 256, False)
    target_embeddings = jnp.concatenate(
        [target_x[target_nodes], pos_enc[target_nodes]], axis=1)

    m0 = (jnp.arange(_NP, dtype=jnp.int32)[:, None]
          == target_nodes[None, :]).astype(jnp.float32)
    m1 = _mm(a_cnt, m0, m0, 'mask', 1024, 64, 512, True)
    m2 = _mm(a_cnt, m1, m1, 'mask', 1024, 64, 512, True)

    aself = _mm(mxp, Wc_self, bc[None, :], 'none', 1024, 128, 128, False)
    y = (m2[:, :, None] * mxp[:, None, :]).reshape(_NP, _T * _D)
    z_td = jnp.zeros((1, _T * _D), jnp.float32)
    g = _mm(at_cnt, y, z_td, 'none', 1024, 512, 512, False)

    m2t = m2.T
    ctx_h = pl.pallas_call(
        _ctx_body,
        grid=(_T // 8, _NP // 1024),
        in_specs=[
            pl.BlockSpec((1024, 8 * _D), lambda tg, v: (v, tg)),
            pl.BlockSpec((1024, _D), lambda tg, v: (v, 0)),
            pl.BlockSpec((8, 1024), lambda tg, v: (tg, v)),
            pl.BlockSpec((_D, _D), lambda tg, v: (0, 0)),
        ],
        out_specs=pl.BlockSpec((8, _D), lambda tg, v: (tg, 0)),
        out_shape=jax.ShapeDtypeStruct((_T, _D), jnp.float32),
    )(g, aself, m2t, Wc_nbr)

    pospad = jnp.concatenate(
        [jnp.pad(pos_enc, ((0, pad), (0, 0))),
         jnp.pad(jnp.ones((_N, 1), jnp.float32), ((0, pad), (0, 0))),
         jnp.zeros((_NP, 128 - _PE - 1), jnp.float32)], axis=1)
    cp = _mm(m2t, pospad, z128, 'none', 64, 128, 1024, False)
    cnt = cp[:, _PE:_PE + 1]
    ctx = jnp.concatenate([ctx_h, cp[:, :_PE]], axis=1) / cnt

    pred = pl.pallas_call(
        _pred_body,
        out_shape=jax.ShapeDtypeStruct((_T, _EMB), jnp.float32),
    )(ctx, Wp1, bp1[None, :], Wp2, bp2[None, :])
    return pred, target_embeddings
